# Initial kernel scaffold; baseline (speedup 1.0000x reference)
import jax
import jax.numpy as jnp
from jax.experimental import pallas as pl


def kernel(x, h_node, batch, W_enc, b_enc, W_gate, b_gate):
    raise NotImplementedError("write your pallas kernel here")


# speedup vs baseline: 2.3679x; 2.3679x over previous
import jax
import jax.numpy as jnp
import numpy as np
from jax.experimental import pallas as pl
from jax.experimental.pallas import tpu as pltpu

N = 100000
D = 128
S = 1024
BLK = 2048
NBLK = (N + BLK - 1) // BLK
NPAD = NBLK * BLK


def _body(x_ref, h_ref, b_ref, wenc_ref, benc_ref, wgate_ref, bgate_ref,
          gate_ref, acc_ref):
    i = pl.program_id(0)

    @pl.when(i == 0)
    def _init():
        acc_ref[...] = jnp.zeros_like(acc_ref)

    xb = x_ref[...]
    hb = h_ref[...]
    enc = jnp.dot(xb, wenc_ref[...], preferred_element_type=jnp.float32,
                  precision=jax.lax.Precision.HIGHEST)
    enc = jnp.maximum(enc + benc_ref[...], 0.0)
    logit = jnp.sum(enc * wgate_ref[...], axis=1, keepdims=True) + bgate_ref[...]
    g = jax.nn.sigmoid(logit)
    gate_ref[...] = g

    seg_ids = jax.lax.broadcasted_iota(jnp.int32, (BLK, S), 1)
    oh = (b_ref[0, 0][:, None] == seg_ids).astype(jnp.bfloat16)

    gh = g * hb
    ch = hb - gh
    ones = jnp.ones((BLK, 1), jnp.float32)
    pad = jnp.zeros((BLK, 126), jnp.float32)
    vals = jnp.concatenate([gh, ch, g, ones - g, pad], axis=1).astype(jnp.bfloat16)

    acc_ref[...] += jax.lax.dot_general(
        oh, vals, (((0,), (0,)), ((), ())),
        preferred_element_type=jnp.float32)


def kernel(x, h_node, batch, W_enc, b_enc, W_gate, b_gate):
    pad = NPAD - N
    xp = jnp.pad(x, ((0, pad), (0, 0)))
    hp = jnp.pad(h_node, ((0, pad), (0, 0)))
    bp = jnp.pad(batch, (0, pad), constant_values=S).reshape(NBLK, 1, BLK)

    gate_p, acc = pl.pallas_call(
        _body,
        grid=(NBLK,),
        in_specs=[
            pl.BlockSpec((BLK, D), lambda i: (i, 0)),
            pl.BlockSpec((BLK, D), lambda i: (i, 0)),
            pl.BlockSpec((1, 1, BLK), lambda i: (i, 0, 0)),
            pl.BlockSpec((D, D), lambda i: (0, 0)),
            pl.BlockSpec((1, D), lambda i: (0, 0)),
            pl.BlockSpec((1, D), lambda i: (0, 0)),
            pl.BlockSpec((1, 1), lambda i: (0, 0)),
        ],
        out_specs=[
            pl.BlockSpec((BLK, 1), lambda i: (i, 0)),
            pl.BlockSpec((S, 384), lambda i: (0, 0)),
        ],
        out_shape=[
            jax.ShapeDtypeStruct((NPAD, 1), jnp.float32),
            jax.ShapeDtypeStruct((S, 384), jnp.float32),
        ],
    )(xp, hp, bp, W_enc, b_enc.reshape(1, D), W_gate.reshape(1, D),
      b_gate.reshape(1, 1))

    gate = gate_p[:N]
    h_out = acc[:, :D]
    c_out = acc[:, D:2 * D]
    rationale_size = acc[:, 2 * D:2 * D + 1] + 1e-08
    envir_size = acc[:, 2 * D + 1:2 * D + 2] + 1e-08
    return (h_out, c_out, rationale_size, envir_size, gate)


# speedup vs baseline: 2.7695x; 1.1696x over previous
import functools

import jax
import jax.numpy as jnp
from jax import lax
from jax.experimental import pallas as pl
from jax.experimental.pallas import tpu as pltpu
from jax.experimental.pallas import tpu_sc as plsc

N = 100000
D = 128
S = 1024

BLK = 2048
NBLK = (N + BLK - 1) // BLK
NPAD = NBLK * BLK

NW = 32
RPW = 3128
CH = 128
MAXF = 1024
SLOTW = 272


def _gate_body(x_ref, wenc_ref, benc_ref, wgate_ref, bgate_ref, gate_ref):
    enc = jnp.dot(x_ref[...], wenc_ref[...], preferred_element_type=jnp.float32)
    enc = jnp.maximum(enc + benc_ref[...], 0.0)
    logit = jnp.sum(enc * wgate_ref[...], axis=1, keepdims=True) + bgate_ref[...]
    gate_ref[...] = jax.nn.sigmoid(logit)


def _gate_pass(x, W_enc, b_enc, W_gate, b_gate):
    return pl.pallas_call(
        _gate_body,
        grid=(NBLK,),
        in_specs=[
            pl.BlockSpec((BLK, D), lambda i: (i, 0)),
            pl.BlockSpec((D, D), lambda i: (0, 0)),
            pl.BlockSpec((1, D), lambda i: (0, 0)),
            pl.BlockSpec((1, D), lambda i: (0, 0)),
            pl.BlockSpec((1, 1), lambda i: (0, 0)),
        ],
        out_specs=pl.BlockSpec((BLK, 1), lambda i: (i, 0)),
        out_shape=jax.ShapeDtypeStruct((NPAD, 1), jnp.float32),
    )(x, W_enc, b_enc.reshape(1, D), W_gate.reshape(1, D),
      b_gate.reshape(1, 1))


def _sc_seg_body(h_hbm, g_hbm, b_hbm, slots_hbm, segs_hbm, cnts_hbm,
                 hbuf, gbuf, bbuf, stage, segbuf, cntbuf):
    w = lax.axis_index("s") * 2 + lax.axis_index("c")
    row0 = w * RPW
    nrows = jnp.minimum(RPW, N - row0)
    nfull = nrows // CH
    ntail = (nrows - nfull * CH) // 8
    lane = lax.iota(jnp.int32, 16)

    zv16 = jnp.zeros((16,), jnp.float32)

    def do_flush(cur, fcnt, gsum, csum):
        tailv = jnp.where(lane == 0, gsum,
                          jnp.where(lane == 1, csum, 0.0))
        stage[pl.ds(256, 16)] = tailv
        segbuf[pl.ds(fcnt, 16)] = jnp.full((16,), cur, jnp.int32)
        pltpu.sync_copy(stage, slots_hbm.at[w].at[fcnt])

    def row_step(carry, s, g, hs):
        (cur, fcnt, gsum, csum) = carry
        changed = s != cur
        doflush = changed & (cur >= 0)

        @pl.when(doflush)
        def _():
            do_flush(cur, fcnt, gsum, csum)
            for j in range(16):
                stage[pl.ds(16 * j, 16)] = zv16

        gv = jnp.full((16,), g, jnp.float32)
        for j in range(8):
            plsc.addupdate(stage.at[pl.ds(16 * j, 16)], gv * hs[j])
            plsc.addupdate(stage.at[pl.ds(128 + 16 * j, 16)], hs[j])
        gsum2 = jnp.where(changed, g, gsum + g)
        csum2 = jnp.where(changed, 1.0, csum + 1.0)
        fcnt2 = fcnt + doflush.astype(jnp.int32)
        return (s, fcnt2, gsum2, csum2)

    def group_step(i0, carry, nr):
        bvec = bbuf[pl.ds(i0, 16)]
        gvec = gbuf[pl.ds(i0, 16)]
        for r in range(nr):
            hs = [hbuf[i0 + r, pl.ds(16 * j, 16)] for j in range(8)]
            carry = row_step(carry, bvec[r], gvec[r], hs)
        return carry

    def chunk_body(k, carry):
        base = row0 + k * CH
        pltpu.sync_copy(h_hbm.at[pl.ds(base, CH)], hbuf)
        pltpu.sync_copy(g_hbm.at[pl.ds(base, CH)], gbuf)
        pltpu.sync_copy(b_hbm.at[pl.ds(base, CH)], bbuf)
        return lax.fori_loop(
            0, CH // 16, lambda t, c: group_step(t * 16, c, 16), carry)

    def tail_body(k, carry):
        base = row0 + nfull * CH + k * 8
        pltpu.sync_copy(h_hbm.at[pl.ds(base, 8)], hbuf.at[pl.ds(0, 8)])
        pltpu.sync_copy(g_hbm.at[pl.ds(base, 8)], gbuf.at[pl.ds(0, 8)])
        pltpu.sync_copy(b_hbm.at[pl.ds(base, 8)], bbuf.at[pl.ds(0, 8)])
        return group_step(0, carry, 8)

    for j in range(16):
        stage[pl.ds(16 * j, 16)] = zv16
    carry0 = (jnp.int32(-1), jnp.int32(0), jnp.float32(0.0),
              jnp.float32(0.0))
    carry = lax.fori_loop(0, nfull, chunk_body, carry0)
    carry = lax.fori_loop(0, ntail, tail_body, carry)

    (cur, fcnt, gsum, csum) = carry

    @pl.when(cur >= 0)
    def _():
        do_flush(cur, fcnt, gsum, csum)

    fin = jnp.where(cur >= 0, fcnt + 1, fcnt)
    cntbuf[pl.ds(0, 16)] = jnp.full((16,), fin, jnp.int32)
    pltpu.sync_copy(segbuf.at[pl.ds(0, MAXF)], segs_hbm.at[w])
    pltpu.sync_copy(cntbuf, cnts_hbm.at[w])


def _sc_segment_sums(h_node, gate, batch):
    mesh = plsc.VectorSubcoreMesh(core_axis_name="c", subcore_axis_name="s")
    kern = pl.kernel(
        _sc_seg_body,
        mesh=mesh,
        out_type=[
            jax.ShapeDtypeStruct((NW, MAXF, SLOTW), jnp.float32),
            jax.ShapeDtypeStruct((NW, MAXF), jnp.int32),
            jax.ShapeDtypeStruct((NW, 16), jnp.int32),
        ],
        scratch_types=[
            pltpu.VMEM((CH, D), jnp.float32),
            pltpu.VMEM((CH,), jnp.float32),
            pltpu.VMEM((CH,), jnp.int32),
            pltpu.VMEM((SLOTW,), jnp.float32),
            pltpu.VMEM((MAXF + 16,), jnp.int32),
            pltpu.VMEM((16,), jnp.int32),
        ],
    )
    return kern(h_node, gate.reshape(-1), batch)


def _combine_body(cnts_ref, slots_hbm, segs_hbm, h_ref, c_ref, r_ref, e_ref,
                  stage, segs_smem, acc, sem0, sem1):
    acc[...] = jnp.zeros_like(acc)

    def worker(w, _):
        cp0 = pltpu.make_async_copy(slots_hbm.at[w], stage, sem0)
        cp1 = pltpu.make_async_copy(segs_hbm.at[w], segs_smem, sem1)
        cp0.start()
        cp1.start()
        cp0.wait()
        cp1.wait()
        cnt = cnts_ref[w, 0]

        def slot(j, _):
            seg = segs_smem[j]
            acc[pl.ds(seg, 1), :] += stage[pl.ds(j, 1), :]
            return 0

        lax.fori_loop(0, cnt, slot, 0)
        return 0

    lax.fori_loop(0, NW, worker, 0)

    hsum = acc[:, :D]
    tsum = acc[:, D:2 * D]
    rat = acc[:, 256:257]
    cnt = acc[:, 257:258]
    h_ref[...] = hsum
    c_ref[...] = tsum - hsum
    r_ref[...] = rat + 1e-08
    e_ref[...] = (cnt - rat) + 1e-08


def _combine(slots, segs, cnts):
    return pl.pallas_call(
        _combine_body,
        in_specs=[
            pl.BlockSpec(memory_space=pltpu.MemorySpace.SMEM),
            pl.BlockSpec(memory_space=pltpu.MemorySpace.HBM),
            pl.BlockSpec(memory_space=pltpu.MemorySpace.HBM),
        ],
        out_specs=[
            pl.BlockSpec((S, D), lambda: (0, 0)),
            pl.BlockSpec((S, D), lambda: (0, 0)),
            pl.BlockSpec((S, 1), lambda: (0, 0)),
            pl.BlockSpec((S, 1), lambda: (0, 0)),
        ],
        out_shape=[
            jax.ShapeDtypeStruct((S, D), jnp.float32),
            jax.ShapeDtypeStruct((S, D), jnp.float32),
            jax.ShapeDtypeStruct((S, 1), jnp.float32),
            jax.ShapeDtypeStruct((S, 1), jnp.float32),
        ],
        scratch_shapes=[
            pltpu.VMEM((MAXF, SLOTW), jnp.float32),
            pltpu.SMEM((MAXF,), jnp.int32),
            pltpu.VMEM((S, SLOTW), jnp.float32),
            pltpu.SemaphoreType.DMA,
            pltpu.SemaphoreType.DMA,
        ],
    )(cnts, slots, segs)


def kernel(x, h_node, batch, W_enc, b_enc, W_gate, b_gate):
    gate_p = _gate_pass(x, W_enc, b_enc, W_gate, b_gate)
    slots, segs, cnts = _sc_segment_sums(h_node, gate_p, batch)
    h_out, c_out, rat, env = _combine(slots, segs, cnts)
    return (h_out, c_out, rat, env, gate_p[:N])


# speedup vs baseline: 3.5746x; 1.2907x over previous
import functools

import jax
import jax.numpy as jnp
from jax import lax
from jax.experimental import pallas as pl
from jax.experimental.pallas import tpu as pltpu
from jax.experimental.pallas import tpu_sc as plsc

N = 100000
D = 128
S = 1024

BLK = 2048
NBLK = (N + BLK - 1) // BLK
NPAD = NBLK * BLK

NW = 32
CH = 256
FULLC = 12
RPW = FULLC * CH
TAILBASE = NW * RPW
TAILU = (N - TAILBASE) // 16
MAXF = 1152
SLOTW = 272


def _gate_body(x_ref, wenc_ref, benc_ref, wgate_ref, bgate_ref, gate_ref):
    enc = jnp.dot(x_ref[...], wenc_ref[...], preferred_element_type=jnp.float32)
    enc = jnp.maximum(enc + benc_ref[...], 0.0)
    logit = jnp.sum(enc * wgate_ref[...], axis=1, keepdims=True) + bgate_ref[...]
    gate_ref[...] = jax.nn.sigmoid(logit)


def _gate_pass(x, W_enc, b_enc, W_gate, b_gate):
    return pl.pallas_call(
        _gate_body,
        grid=(NBLK,),
        in_specs=[
            pl.BlockSpec((BLK, D), lambda i: (i, 0)),
            pl.BlockSpec((D, D), lambda i: (0, 0)),
            pl.BlockSpec((1, D), lambda i: (0, 0)),
            pl.BlockSpec((1, D), lambda i: (0, 0)),
            pl.BlockSpec((1, 1), lambda i: (0, 0)),
        ],
        out_specs=pl.BlockSpec((BLK, 1), lambda i: (i, 0)),
        out_shape=jax.ShapeDtypeStruct((NPAD, 1), jnp.float32),
    )(x, W_enc, b_enc.reshape(1, D), W_gate.reshape(1, D),
      b_gate.reshape(1, 1))


def _sc_seg_body(h_hbm, g_hbm, b_hbm, slots_hbm, segs_hbm, cnts_hbm,
                 hbufA, gbufA, bbufA, hbufB, gbufB, bbufB,
                 stage, segbuf, cntbuf,
                 semhA, semgA, sembA, semhB, semgB, sembB):
    w = lax.axis_index("s") * 2 + lax.axis_index("c")
    row0 = w * RPW
    lane = lax.iota(jnp.int32, 16)

    def dma3(base, nr, hb, gb, bb, sh, sg, sb):
        return (
            pltpu.make_async_copy(h_hbm.at[pl.ds(base, nr)],
                                  hb.at[pl.ds(0, nr)], sh),
            pltpu.make_async_copy(g_hbm.at[pl.ds(base, nr)],
                                  gb.at[pl.ds(0, nr)], sg),
            pltpu.make_async_copy(b_hbm.at[pl.ds(base, nr)],
                                  bb.at[pl.ds(0, nr)], sb),
        )

    def start3(base, nr, hb, gb, bb, sh, sg, sb):
        for cp in dma3(base, nr, hb, gb, bb, sh, sg, sb):
            cp.start()

    def wait3(base, nr, hb, gb, bb, sh, sg, sb):
        for cp in dma3(base, nr, hb, gb, bb, sh, sg, sb):
            cp.wait()

    def do_flush(cur, fcnt, gsum, rowidx, vs, ts):
        for j in range(8):
            stage[pl.ds(16 * j, 16)] = vs[j]
            stage[pl.ds(128 + 16 * j, 16)] = ts[j]
        tailv = jnp.where(lane == 0, gsum,
                          jnp.where(lane == 1, rowidx.astype(jnp.float32),
                                    0.0))
        stage[pl.ds(256, 16)] = tailv
        segbuf[pl.ds(fcnt, 16)] = jnp.full((16,), cur, jnp.int32)
        pltpu.sync_copy(stage, slots_hbm.at[w].at[fcnt])

    def group_step(hb, gb, bb, i0, lbase, carry):
        (cur, fcnt, gsum) = carry[:3]
        vs = list(carry[3:11])
        ts = list(carry[11:19])
        bvec = bb[pl.ds(i0, 16)]
        gvec = gb[pl.ds(i0, 16)]
        for r in range(16):
            s = bvec[r]
            g = gvec[r]
            doflush = (s != cur) & (cur >= 0)

            @pl.when(doflush)
            def _(cur=cur, fcnt=fcnt, gsum=gsum, ridx=lbase + r,
                  vs=tuple(vs), ts=tuple(ts)):
                do_flush(cur, fcnt, gsum, ridx, vs, ts)

            fcnt = fcnt + doflush.astype(jnp.int32)
            cur = s
            gsum = gsum + g
            gv = jnp.full((16,), g, jnp.float32)
            for j in range(8):
                hsj = hb[i0 + r, pl.ds(16 * j, 16)]
                vs[j] = vs[j] + gv * hsj
                ts[j] = ts[j] + hsj
        return (cur, fcnt, gsum) + tuple(vs) + tuple(ts)

    def process(hb, gb, bb, lbase, carry):
        def grp(t, c):
            return group_step(hb, gb, bb, t * 16, lbase + t * 16, c)
        return lax.fori_loop(0, CH // 16, grp, carry)

    zv16 = jnp.zeros((16,), jnp.float32)
    carry = (jnp.int32(-1), jnp.int32(0), jnp.float32(0.0)) + (zv16,) * 16

    start3(row0, CH, hbufA, gbufA, bbufA, semhA, semgA, sembA)

    def pair(t, carry):
        k0 = 2 * t
        b0 = row0 + k0 * CH
        b1 = b0 + CH
        wait3(b0, CH, hbufA, gbufA, bbufA, semhA, semgA, sembA)
        start3(b1, CH, hbufB, gbufB, bbufB, semhB, semgB, sembB)
        carry = process(hbufA, gbufA, bbufA, k0 * CH, carry)
        wait3(b1, CH, hbufB, gbufB, bbufB, semhB, semgB, sembB)

        @pl.when(t < FULLC // 2 - 1)
        def _():
            start3(b1 + CH, CH, hbufA, gbufA, bbufA, semhA, semgA, sembA)

        return process(hbufB, gbufB, bbufB, (k0 + 1) * CH, carry)

    carry = lax.fori_loop(0, FULLC // 2, pair, carry)

    (cur, fcnt, gsum) = carry[:3]

    @pl.when(cur >= 0)
    def _():
        do_flush(cur, fcnt, gsum, jnp.int32(RPW), carry[3:11], carry[11:19])

    fcnt = fcnt + (cur >= 0).astype(jnp.int32)
    carry = (jnp.int32(-1), fcnt, gsum) + carry[3:]

    ntw = jnp.where(w < TAILU - 3 * NW, 4, 3)
    u0 = w * 3 + jnp.minimum(w, TAILU - 3 * NW)

    def tail_unit(u, carry):
        base = TAILBASE + (u0 + u) * 16
        pltpu.sync_copy(h_hbm.at[pl.ds(base, 16)], hbufA.at[pl.ds(0, 16)])
        pltpu.sync_copy(g_hbm.at[pl.ds(base, 16)], gbufA.at[pl.ds(0, 16)])
        pltpu.sync_copy(b_hbm.at[pl.ds(base, 16)], bbufA.at[pl.ds(0, 16)])
        return group_step(hbufA, gbufA, bbufA, 0, RPW + u * 16, carry)

    carry = lax.fori_loop(0, ntw, tail_unit, carry)

    (cur, fcnt, gsum) = carry[:3]

    @pl.when(cur >= 0)
    def _():
        do_flush(cur, fcnt, gsum, RPW + ntw * 16, carry[3:11], carry[11:19])

    fin = fcnt + (cur >= 0).astype(jnp.int32)
    cntbuf[pl.ds(0, 16)] = jnp.full((16,), fin, jnp.int32)
    pltpu.sync_copy(segbuf.at[pl.ds(0, MAXF)], segs_hbm.at[w])
    pltpu.sync_copy(cntbuf, cnts_hbm.at[w])


def _sc_segment_sums(h_node, gate, batch):
    mesh = plsc.VectorSubcoreMesh(core_axis_name="c", subcore_axis_name="s")
    kern = pl.kernel(
        _sc_seg_body,
        mesh=mesh,
        out_type=[
            jax.ShapeDtypeStruct((NW, MAXF, SLOTW), jnp.float32),
            jax.ShapeDtypeStruct((NW, MAXF), jnp.int32),
            jax.ShapeDtypeStruct((NW, 16), jnp.int32),
        ],
        scratch_types=[
            pltpu.VMEM((CH, D), jnp.float32),
            pltpu.VMEM((CH,), jnp.float32),
            pltpu.VMEM((CH,), jnp.int32),
            pltpu.VMEM((CH, D), jnp.float32),
            pltpu.VMEM((CH,), jnp.float32),
            pltpu.VMEM((CH,), jnp.int32),
            pltpu.VMEM((SLOTW,), jnp.float32),
            pltpu.VMEM((MAXF + 16,), jnp.int32),
            pltpu.VMEM((16,), jnp.int32),
            pltpu.SemaphoreType.DMA,
            pltpu.SemaphoreType.DMA,
            pltpu.SemaphoreType.DMA,
            pltpu.SemaphoreType.DMA,
            pltpu.SemaphoreType.DMA,
            pltpu.SemaphoreType.DMA,
        ],
    )
    return kern(h_node, gate.reshape(-1), batch)


def _combine_body(cnts_ref, slots_hbm, segs_hbm, h_ref, c_ref, r_ref, e_ref,
                  stage, segs_smem, acc, sem0, sem1):
    acc[...] = jnp.zeros_like(acc)

    def worker(w, _):
        cp0 = pltpu.make_async_copy(slots_hbm.at[w], stage, sem0)
        cp1 = pltpu.make_async_copy(segs_hbm.at[w], segs_smem, sem1)
        cp0.start()
        cp1.start()
        cp0.wait()
        cp1.wait()
        cnt = cnts_ref[w, 0]

        def slot(j, prev):
            seg = segs_smem[j]
            curv = stage[pl.ds(j, 1), :]
            acc[pl.ds(seg, 1), :] += curv - prev
            return curv

        lax.fori_loop(0, cnt, slot, jnp.zeros((1, SLOTW), jnp.float32))
        return 0

    lax.fori_loop(0, NW, worker, 0)

    hsum = acc[:, :D]
    tsum = acc[:, D:2 * D]
    rat = acc[:, 256:257]
    cnt = acc[:, 257:258]
    h_ref[...] = hsum
    c_ref[...] = tsum - hsum
    r_ref[...] = rat + 1e-08
    e_ref[...] = (cnt - rat) + 1e-08


def _combine(slots, segs, cnts):
    return pl.pallas_call(
        _combine_body,
        in_specs=[
            pl.BlockSpec(memory_space=pltpu.MemorySpace.SMEM),
            pl.BlockSpec(memory_space=pltpu.MemorySpace.HBM),
            pl.BlockSpec(memory_space=pltpu.MemorySpace.HBM),
        ],
        out_specs=[
            pl.BlockSpec((S, D), lambda: (0, 0)),
            pl.BlockSpec((S, D), lambda: (0, 0)),
            pl.BlockSpec((S, 1), lambda: (0, 0)),
            pl.BlockSpec((S, 1), lambda: (0, 0)),
        ],
        out_shape=[
            jax.ShapeDtypeStruct((S, D), jnp.float32),
            jax.ShapeDtypeStruct((S, D), jnp.float32),
            jax.ShapeDtypeStruct((S, 1), jnp.float32),
            jax.ShapeDtypeStruct((S, 1), jnp.float32),
        ],
        scratch_shapes=[
            pltpu.VMEM((MAXF, SLOTW), jnp.float32),
            pltpu.SMEM((MAXF,), jnp.int32),
            pltpu.VMEM((S, SLOTW), jnp.float32),
            pltpu.SemaphoreType.DMA,
            pltpu.SemaphoreType.DMA,
        ],
    )(cnts, slots, segs)


def kernel(x, h_node, batch, W_enc, b_enc, W_gate, b_gate):
    gate_p = _gate_pass(x, W_enc, b_enc, W_gate, b_gate)
    slots, segs, cnts = _sc_segment_sums(h_node, gate_p, batch)
    h_out, c_out, rat, env = _combine(slots, segs, cnts)
    return (h_out, c_out, rat, env, gate_p[:N])


# speedup vs baseline: 3.9910x; 1.1165x over previous
import functools

import jax
import jax.numpy as jnp
from jax import lax
from jax.experimental import pallas as pl
from jax.experimental.pallas import tpu as pltpu
from jax.experimental.pallas import tpu_sc as plsc

N = 100000
D = 128
S = 1024

BLK = 2048
NBLK = (N + BLK - 1) // BLK
NPAD = NBLK * BLK

NW = 32
CH = 256
FULLC = 12
RPW = FULLC * CH
TAILBASE = NW * RPW
TAILU = (N - TAILBASE) // 16
MAXF = 1152
SLOTW = 272


def _gate_body(x_ref, wenc_ref, benc_ref, wgate_ref, bgate_ref, gate_ref):
    enc = jnp.dot(x_ref[...], wenc_ref[...], preferred_element_type=jnp.float32)
    enc = jnp.maximum(enc + benc_ref[...], 0.0)
    logit = jnp.sum(enc * wgate_ref[...], axis=1, keepdims=True) + bgate_ref[...]
    gate_ref[...] = jax.nn.sigmoid(logit)


def _gate_pass(x, W_enc, b_enc, W_gate, b_gate):
    return pl.pallas_call(
        _gate_body,
        grid=(NBLK,),
        in_specs=[
            pl.BlockSpec((BLK, D), lambda i: (i, 0)),
            pl.BlockSpec((D, D), lambda i: (0, 0)),
            pl.BlockSpec((1, D), lambda i: (0, 0)),
            pl.BlockSpec((1, D), lambda i: (0, 0)),
            pl.BlockSpec((1, 1), lambda i: (0, 0)),
        ],
        out_specs=pl.BlockSpec((BLK, 1), lambda i: (i, 0)),
        out_shape=jax.ShapeDtypeStruct((NPAD, 1), jnp.float32),
    )(x, W_enc, b_enc.reshape(1, D), W_gate.reshape(1, D),
      b_gate.reshape(1, 1))


def _sc_seg_body(h_hbm, g_hbm, b_hbm, slots_hbm, segs_hbm, cnts_hbm,
                 hbufA, gbufA, bbufA, hbufB, gbufB, bbufB,
                 stage, segbuf, cntbuf,
                 semhA, semgA, sembA, semhB, semgB, sembB):
    w = lax.axis_index("s") * 2 + lax.axis_index("c")
    row0 = w * RPW
    lane = lax.iota(jnp.int32, 16)

    def dma3(base, nr, hb, gb, bb, sh, sg, sb):
        return (
            pltpu.make_async_copy(h_hbm.at[pl.ds(base, nr)],
                                  hb.at[pl.ds(0, nr)], sh),
            pltpu.make_async_copy(g_hbm.at[pl.ds(base, nr)],
                                  gb.at[pl.ds(0, nr)], sg),
            pltpu.make_async_copy(b_hbm.at[pl.ds(base, nr)],
                                  bb.at[pl.ds(0, nr)], sb),
        )

    def start3(base, nr, hb, gb, bb, sh, sg, sb):
        for cp in dma3(base, nr, hb, gb, bb, sh, sg, sb):
            cp.start()

    def wait3(base, nr, hb, gb, bb, sh, sg, sb):
        for cp in dma3(base, nr, hb, gb, bb, sh, sg, sb):
            cp.wait()

    def do_flush(cur, fcnt, gsum, rowidx, vs, ts):
        for j in range(8):
            stage[pl.ds(16 * j, 16)] = vs[j]
            stage[pl.ds(128 + 16 * j, 16)] = ts[j]
        tailv = jnp.where(lane == 0, gsum,
                          jnp.where(lane == 1, rowidx.astype(jnp.float32),
                                    0.0))
        stage[pl.ds(256, 16)] = tailv
        segbuf[pl.ds(fcnt, 16)] = jnp.full((16,), cur, jnp.int32)
        pltpu.sync_copy(stage, slots_hbm.at[w].at[fcnt])

    def group_step(hb, gb, bb, i0, lbase, carry):
        (cur, fcnt, gsum) = carry[:3]
        vs = list(carry[3:11])
        ts = list(carry[11:19])
        bvec = bb[pl.ds(i0, 16)]
        gvec = gb[pl.ds(i0, 16)]
        for r in range(16):
            s = bvec[r]
            g = gvec[r]
            doflush = (s != cur) & (cur >= 0)

            @pl.when(doflush)
            def _(cur=cur, fcnt=fcnt, gsum=gsum, ridx=lbase + r,
                  vs=tuple(vs), ts=tuple(ts)):
                do_flush(cur, fcnt, gsum, ridx, vs, ts)

            fcnt = fcnt + doflush.astype(jnp.int32)
            cur = s
            gsum = gsum + g
            gv = jnp.full((16,), g, jnp.float32)
            for j in range(8):
                hsj = hb[i0 + r, pl.ds(16 * j, 16)]
                vs[j] = vs[j] + gv * hsj
                ts[j] = ts[j] + hsj
        return (cur, fcnt, gsum) + tuple(vs) + tuple(ts)

    def process(hb, gb, bb, lbase, carry):
        def grp(t, c):
            return group_step(hb, gb, bb, t * 16, lbase + t * 16, c)
        return lax.fori_loop(0, CH // 16, grp, carry)

    zv16 = jnp.zeros((16,), jnp.float32)
    carry = (jnp.int32(-1), jnp.int32(0), jnp.float32(0.0)) + (zv16,) * 16

    start3(row0, CH, hbufA, gbufA, bbufA, semhA, semgA, sembA)

    def pair(t, carry):
        k0 = 2 * t
        b0 = row0 + k0 * CH
        b1 = b0 + CH
        wait3(b0, CH, hbufA, gbufA, bbufA, semhA, semgA, sembA)
        start3(b1, CH, hbufB, gbufB, bbufB, semhB, semgB, sembB)
        carry = process(hbufA, gbufA, bbufA, k0 * CH, carry)
        wait3(b1, CH, hbufB, gbufB, bbufB, semhB, semgB, sembB)

        @pl.when(t < FULLC // 2 - 1)
        def _():
            start3(b1 + CH, CH, hbufA, gbufA, bbufA, semhA, semgA, sembA)

        return process(hbufB, gbufB, bbufB, (k0 + 1) * CH, carry)

    carry = lax.fori_loop(0, FULLC // 2, pair, carry)

    (cur, fcnt, gsum) = carry[:3]

    @pl.when(cur >= 0)
    def _():
        do_flush(cur, fcnt, gsum, jnp.int32(RPW), carry[3:11], carry[11:19])

    fcnt = fcnt + (cur >= 0).astype(jnp.int32)
    carry = (jnp.int32(-1), fcnt, gsum) + carry[3:]

    ntw = jnp.where(w < TAILU - 3 * NW, 4, 3)
    u0 = w * 3 + jnp.minimum(w, TAILU - 3 * NW)

    def tail_unit(u, carry):
        base = TAILBASE + (u0 + u) * 16
        pltpu.sync_copy(h_hbm.at[pl.ds(base, 16)], hbufA.at[pl.ds(0, 16)])
        pltpu.sync_copy(g_hbm.at[pl.ds(base, 16)], gbufA.at[pl.ds(0, 16)])
        pltpu.sync_copy(b_hbm.at[pl.ds(base, 16)], bbufA.at[pl.ds(0, 16)])
        return group_step(hbufA, gbufA, bbufA, 0, RPW + u * 16, carry)

    carry = lax.fori_loop(0, ntw, tail_unit, carry)

    (cur, fcnt, gsum) = carry[:3]

    @pl.when(cur >= 0)
    def _():
        do_flush(cur, fcnt, gsum, RPW + ntw * 16, carry[3:11], carry[11:19])

    fin = fcnt + (cur >= 0).astype(jnp.int32)
    cntbuf[pl.ds(0, 16)] = jnp.full((16,), fin, jnp.int32)
    pltpu.sync_copy(segbuf.at[pl.ds(0, MAXF)], segs_hbm.at[w])
    pltpu.sync_copy(cntbuf, cnts_hbm.at[w])


def _sc_segment_sums(h_node, gate, batch):
    mesh = plsc.VectorSubcoreMesh(core_axis_name="c", subcore_axis_name="s")
    kern = pl.kernel(
        _sc_seg_body,
        mesh=mesh,
        out_type=[
            jax.ShapeDtypeStruct((NW, MAXF, SLOTW), jnp.float32),
            jax.ShapeDtypeStruct((NW, MAXF), jnp.int32),
            jax.ShapeDtypeStruct((NW, 16), jnp.int32),
        ],
        scratch_types=[
            pltpu.VMEM((CH, D), jnp.float32),
            pltpu.VMEM((CH,), jnp.float32),
            pltpu.VMEM((CH,), jnp.int32),
            pltpu.VMEM((CH, D), jnp.float32),
            pltpu.VMEM((CH,), jnp.float32),
            pltpu.VMEM((CH,), jnp.int32),
            pltpu.VMEM((SLOTW,), jnp.float32),
            pltpu.VMEM((MAXF + 16,), jnp.int32),
            pltpu.VMEM((16,), jnp.int32),
            pltpu.SemaphoreType.DMA,
            pltpu.SemaphoreType.DMA,
            pltpu.SemaphoreType.DMA,
            pltpu.SemaphoreType.DMA,
            pltpu.SemaphoreType.DMA,
            pltpu.SemaphoreType.DMA,
        ],
    )
    return kern(h_node, gate.reshape(-1), batch)


def _combine_body(cnts_ref, slots_ref, segs_hbm, h_ref, c_ref, r_ref, e_ref,
                  smemA, smemB, acc, semA, semB):
    w = pl.program_id(0)

    @pl.when(w == 0)
    def _():
        acc[...] = jnp.zeros_like(acc)
        pltpu.make_async_copy(segs_hbm.at[0], smemA, semA).start()
        pltpu.make_async_copy(segs_hbm.at[0], smemA, semA).wait()

    even = (w % 2) == 0

    @pl.when(even & (w + 1 < NW))
    def _():
        pltpu.make_async_copy(segs_hbm.at[w + 1], smemB, semB).start()

    @pl.when((~even) & (w + 1 < NW))
    def _():
        pltpu.make_async_copy(segs_hbm.at[w + 1], smemA, semA).start()

    def process(seg_smem):
        cnt = cnts_ref[w, 0]

        def slot(j, prev):
            seg = seg_smem[j]
            curv = slots_ref[0, pl.ds(j, 1), :]
            acc[pl.ds(seg, 1), :] += curv - prev
            return curv

        lax.fori_loop(0, cnt, slot, jnp.zeros((1, SLOTW), jnp.float32))

    @pl.when(even)
    def _():
        @pl.when(w > 0)
        def _():
            pltpu.make_async_copy(segs_hbm.at[w], smemA, semA).wait()
        process(smemA)

    @pl.when(~even)
    def _():
        pltpu.make_async_copy(segs_hbm.at[w], smemB, semB).wait()
        process(smemB)

    @pl.when(w == NW - 1)
    def _():
        hsum = acc[:, :D]
        tsum = acc[:, D:2 * D]
        rat = acc[:, 256:257]
        cnt = acc[:, 257:258]
        h_ref[...] = hsum
        c_ref[...] = tsum - hsum
        r_ref[...] = rat + 1e-08
        e_ref[...] = (cnt - rat) + 1e-08


def _combine(slots, segs, cnts):
    return pl.pallas_call(
        _combine_body,
        grid=(NW,),
        in_specs=[
            pl.BlockSpec(memory_space=pltpu.MemorySpace.SMEM),
            pl.BlockSpec((1, MAXF, SLOTW), lambda w: (w, 0, 0)),
            pl.BlockSpec(memory_space=pltpu.MemorySpace.HBM),
        ],
        out_specs=[
            pl.BlockSpec((S, D), lambda w: (0, 0)),
            pl.BlockSpec((S, D), lambda w: (0, 0)),
            pl.BlockSpec((S, 1), lambda w: (0, 0)),
            pl.BlockSpec((S, 1), lambda w: (0, 0)),
        ],
        out_shape=[
            jax.ShapeDtypeStruct((S, D), jnp.float32),
            jax.ShapeDtypeStruct((S, D), jnp.float32),
            jax.ShapeDtypeStruct((S, 1), jnp.float32),
            jax.ShapeDtypeStruct((S, 1), jnp.float32),
        ],
        scratch_shapes=[
            pltpu.SMEM((MAXF,), jnp.int32),
            pltpu.SMEM((MAXF,), jnp.int32),
            pltpu.VMEM((S, SLOTW), jnp.float32),
            pltpu.SemaphoreType.DMA,
            pltpu.SemaphoreType.DMA,
        ],
    )(cnts, slots, segs)


def kernel(x, h_node, batch, W_enc, b_enc, W_gate, b_gate):
    gate_p = _gate_pass(x, W_enc, b_enc, W_gate, b_gate)
    slots, segs, cnts = _sc_segment_sums(h_node, gate_p, batch)
    h_out, c_out, rat, env = _combine(slots, segs, cnts)
    return (h_out, c_out, rat, env, gate_p[:N])
